# Initial kernel scaffold; baseline (speedup 1.0000x reference)
import jax
import jax.numpy as jnp
from jax.experimental import pallas as pl


def kernel(hidden_states, ln1_weight, ln1_bias, ln2_weight, ln2_bias, qkv_weight, proj_weight, router_weight, moe_w1, moe_w2):
    raise NotImplementedError("write your pallas kernel here")


# speedup vs baseline: 8.0750x; 8.0750x over previous
import functools
import jax
import jax.numpy as jnp
from jax import lax
from jax.experimental import pallas as pl
from jax.experimental.pallas import tpu as pltpu

NH = 16
TOPK = 2
TM = 256
MT = 128


def _k1_ln_qkv(x_ref, w_ref, g_ref, b_ref, o_ref):
    x = x_ref[...]
    mu = jnp.mean(x, axis=-1, keepdims=True)
    var = jnp.mean((x - mu) ** 2, axis=-1, keepdims=True)
    ln = (x - mu) / jnp.sqrt(var + 1e-5) * g_ref[...] + b_ref[...]
    o_ref[...] = lax.dot_general(ln, w_ref[...], (((1,), (1,)), ((), ())),
                                 preferred_element_type=jnp.float32)


def _k2_attn(q_ref, k_ref, v_ref, o_ref, *, scale, hd):
    S = q_ref.shape[0]
    ri = lax.broadcasted_iota(jnp.int32, (S, S), 0)
    ci = lax.broadcasted_iota(jnp.int32, (S, S), 1)
    causal = ci <= ri
    for j in range(q_ref.shape[1] // hd):
        sl = slice(j * hd, (j + 1) * hd)
        q = q_ref[:, sl]
        k = k_ref[:, sl]
        v = v_ref[:, sl]
        s = lax.dot_general(q, k, (((1,), (1,)), ((), ())),
                            preferred_element_type=jnp.float32) * scale
        s = jnp.where(causal, s, jnp.float32(-1e9))
        m = jnp.max(s, axis=-1, keepdims=True)
        e = jnp.exp(s - m)
        p = e / jnp.sum(e, axis=-1, keepdims=True)
        o_ref[:, sl] = lax.dot_general(p, v, (((1,), (0,)), ((), ())),
                                       preferred_element_type=jnp.float32)


def _k3_proj_router(attn_ref, pw_ref, x_ref, g_ref, b_ref, rw_ref,
                    h_ref, ln2_ref, tp_ref, ti_ref):
    h = x_ref[...] + lax.dot_general(attn_ref[...], pw_ref[...],
                                     (((1,), (1,)), ((), ())),
                                     preferred_element_type=jnp.float32)
    h_ref[...] = h
    mu = jnp.mean(h, axis=-1, keepdims=True)
    var = jnp.mean((h - mu) ** 2, axis=-1, keepdims=True)
    ln = (h - mu) / jnp.sqrt(var + 1e-5) * g_ref[...] + b_ref[...]
    ln2_ref[...] = ln
    logits = lax.dot_general(ln, rw_ref[...], (((1,), (1,)), ((), ())),
                             preferred_element_type=jnp.float32)
    mx = jnp.max(logits, axis=-1, keepdims=True)
    ex = jnp.exp(logits - mx)
    pr = ex / jnp.sum(ex, axis=-1, keepdims=True)
    E = pr.shape[-1]
    idx = lax.broadcasted_iota(jnp.int32, pr.shape, 1)
    p1 = jnp.max(pr, axis=-1, keepdims=True)
    i1 = jnp.min(jnp.where(pr == p1, idx, E), axis=-1, keepdims=True)
    pr2 = jnp.where(idx == i1, jnp.float32(-1.0), pr)
    p2 = jnp.max(pr2, axis=-1, keepdims=True)
    i2 = jnp.min(jnp.where(pr2 == p2, idx, E), axis=-1, keepdims=True)
    z = jnp.zeros((pr.shape[0], 6), jnp.float32)
    tp_ref[...] = jnp.concatenate([p1, p2, z], axis=-1)
    ti_ref[...] = jnp.concatenate([i1, i2, z.astype(jnp.int32)], axis=-1)


def _k5_moe(ue_ref, um_ref, uf_ref, uv_ref, off_ref,
            x_ref, w1_ref, w2_ref, o_ref):
    i = pl.program_id(0)

    @pl.when(uf_ref[i] == 1)
    def _():
        o_ref[...] = jnp.zeros_like(o_ref)

    @pl.when(uv_ref[i] == 1)
    def _():
        x = x_ref[...]
        mid = lax.dot_general(x, w1_ref[0], (((1,), (0,)), ((), ())),
                              preferred_element_type=jnp.float32)
        mid = mid * 0.5 * (1.0 + lax.erf(mid * 0.7071067811865476))
        out = lax.dot_general(mid, w2_ref[0], (((1,), (0,)), ((), ())),
                              preferred_element_type=jnp.float32)
        e = ue_ref[i]
        lo = off_ref[e]
        hi = off_ref[e + 1]
        rows = um_ref[i] * MT + lax.broadcasted_iota(jnp.int32, (MT, 1), 0)
        act = (rows >= lo) & (rows < hi)
        o_ref[...] += jnp.where(act, out, 0.0)


def _k7_combine(h_ref, tp_ref, r_ref, o_ref):
    o_ref[...] = (h_ref[...]
                  + tp_ref[:, 0:1] * r_ref[:, 0, :]
                  + tp_ref[:, 1:2] * r_ref[:, 1, :])


def _gather(table, idx):
    return jnp.take(table, idx, axis=0)


def kernel(hidden_states, ln1_weight, ln1_bias, ln2_weight, ln2_bias,
           qkv_weight, proj_weight, router_weight, moe_w1, moe_w2):
    S, B, H = hidden_states.shape
    E = router_weight.shape[0]
    F = moe_w1.shape[2]
    T = S * B
    P = T * TOPK
    hd = H // NH
    x2d = hidden_states.reshape(T, H)
    g1 = ln1_weight.reshape(1, H)
    b1 = ln1_bias.reshape(1, H)
    g2 = ln2_weight.reshape(1, H)
    b2 = ln2_bias.reshape(1, H)

    qkv = pl.pallas_call(
        _k1_ln_qkv,
        grid=(T // TM,),
        in_specs=[
            pl.BlockSpec((TM, H), lambda i: (i, 0)),
            pl.BlockSpec((3 * H, H), lambda i: (0, 0)),
            pl.BlockSpec((1, H), lambda i: (0, 0)),
            pl.BlockSpec((1, H), lambda i: (0, 0)),
        ],
        out_specs=pl.BlockSpec((TM, 3 * H), lambda i: (i, 0)),
        out_shape=jax.ShapeDtypeStruct((T, 3 * H), jnp.float32),
    )(x2d, qkv_weight, g1, b1)

    hpg = max(1, 128 // hd)
    hb = hpg * hd
    ng = NH // hpg
    attn = pl.pallas_call(
        functools.partial(_k2_attn, scale=1.0 / (hd ** 0.5), hd=hd),
        grid=(ng,),
        in_specs=[
            pl.BlockSpec((T, hb), lambda h: (0, h)),
            pl.BlockSpec((T, hb), lambda h: (0, ng + h)),
            pl.BlockSpec((T, hb), lambda h: (0, 2 * ng + h)),
        ],
        out_specs=pl.BlockSpec((T, hb), lambda h: (0, h)),
        out_shape=jax.ShapeDtypeStruct((T, H), jnp.float32),
    )(qkv, qkv, qkv)

    h, ln2, top_p, top_i = pl.pallas_call(
        _k3_proj_router,
        grid=(T // TM,),
        in_specs=[
            pl.BlockSpec((TM, H), lambda i: (i, 0)),
            pl.BlockSpec((H, H), lambda i: (0, 0)),
            pl.BlockSpec((TM, H), lambda i: (i, 0)),
            pl.BlockSpec((1, H), lambda i: (0, 0)),
            pl.BlockSpec((1, H), lambda i: (0, 0)),
            pl.BlockSpec((E, H), lambda i: (0, 0)),
        ],
        out_specs=[
            pl.BlockSpec((TM, H), lambda i: (i, 0)),
            pl.BlockSpec((TM, H), lambda i: (i, 0)),
            pl.BlockSpec((TM, 8), lambda i: (i, 0)),
            pl.BlockSpec((TM, 8), lambda i: (i, 0)),
        ],
        out_shape=[
            jax.ShapeDtypeStruct((T, H), jnp.float32),
            jax.ShapeDtypeStruct((T, H), jnp.float32),
            jax.ShapeDtypeStruct((T, 8), jnp.float32),
            jax.ShapeDtypeStruct((T, 8), jnp.int32),
        ],
    )(attn, proj_weight, x2d, g2, b2, router_weight)

    flat_expert = top_i[:, :TOPK].reshape(-1)
    sorted_indices = jnp.argsort(flat_expert, stable=True)
    token_ids = sorted_indices // TOPK
    restore = jnp.argsort(sorted_indices)
    sizes = jnp.bincount(flat_expert, length=E)
    off = jnp.concatenate([jnp.zeros(1, jnp.int32),
                           jnp.cumsum(sizes).astype(jnp.int32)])
    n_mt = P // MT
    UNITS = n_mt + E - 1
    t0 = off[:E] // MT
    t1 = jnp.maximum(off[1:] - 1, 0) // MT
    nu = jnp.where(sizes > 0, t1 - t0 + 1, 0).astype(jnp.int32)
    ucum = jnp.concatenate([jnp.zeros(1, jnp.int32),
                            jnp.cumsum(nu).astype(jnp.int32)])
    W = ucum[E]
    ii = jnp.arange(UNITS, dtype=jnp.int32)
    eidx = jnp.clip(jnp.searchsorted(ucum, ii, side='right') - 1, 0, E - 1)
    eidx = eidx.astype(jnp.int32)
    valid = ii < W
    e_last = jnp.max(jnp.where(valid, eidx, -1)).astype(jnp.int32)
    um = jnp.where(valid, t0[eidx] + (ii - ucum[eidx]), n_mt - 1)
    um = jnp.clip(um, 0, n_mt - 1).astype(jnp.int32)
    ue = jnp.where(valid, eidx, e_last).astype(jnp.int32)
    uf = jnp.concatenate([jnp.ones(1, jnp.int32),
                          (um[1:] != um[:-1]).astype(jnp.int32)])
    uv = valid.astype(jnp.int32)

    xg = _gather(ln2, token_ids)

    eout = pl.pallas_call(
        _k5_moe,
        grid_spec=pltpu.PrefetchScalarGridSpec(
            num_scalar_prefetch=5,
            grid=(UNITS,),
            in_specs=[
                pl.BlockSpec((MT, H), lambda i, ue, um, uf, uv, off: (um[i], 0)),
                pl.BlockSpec((1, H, F), lambda i, ue, um, uf, uv, off: (ue[i], 0, 0)),
                pl.BlockSpec((1, F, H), lambda i, ue, um, uf, uv, off: (ue[i], 0, 0)),
            ],
            out_specs=pl.BlockSpec((MT, H), lambda i, ue, um, uf, uv, off: (um[i], 0)),
        ),
        out_shape=jax.ShapeDtypeStruct((P, H), jnp.float32),
    )(ue, um, uf, uv, off, xg, moe_w1, moe_w2)

    r = _gather(eout, restore).reshape(T, TOPK, H)
    out = pl.pallas_call(
        _k7_combine,
        grid=(T // TM,),
        in_specs=[
            pl.BlockSpec((TM, H), lambda i: (i, 0)),
            pl.BlockSpec((TM, 8), lambda i: (i, 0)),
            pl.BlockSpec((TM, TOPK, H), lambda i: (i, 0, 0)),
        ],
        out_specs=pl.BlockSpec((TM, H), lambda i: (i, 0)),
        out_shape=jax.ShapeDtypeStruct((T, H), jnp.float32),
    )(h, top_p, r)
    return out.reshape(S, B, H)


# speedup vs baseline: 8.9346x; 1.1065x over previous
import functools
import jax
import jax.numpy as jnp
from jax import lax
from jax.experimental import pallas as pl
from jax.experimental.pallas import tpu as pltpu
from jax.experimental.pallas import tpu_sc as plsc

NH = 16
TOPK = 2
TM = 256
MT = 128


def _k1_ln_qkv(x_ref, w_ref, g_ref, b_ref, o_ref):
    x = x_ref[...]
    mu = jnp.mean(x, axis=-1, keepdims=True)
    var = jnp.mean((x - mu) ** 2, axis=-1, keepdims=True)
    ln = (x - mu) / jnp.sqrt(var + 1e-5) * g_ref[...] + b_ref[...]
    o_ref[...] = lax.dot_general(ln, w_ref[...], (((1,), (1,)), ((), ())),
                                 preferred_element_type=jnp.float32)


def _k2_attn(q_ref, k_ref, v_ref, o_ref, *, scale, hd):
    S = q_ref.shape[0]
    ri = lax.broadcasted_iota(jnp.int32, (S, S), 0)
    ci = lax.broadcasted_iota(jnp.int32, (S, S), 1)
    causal = ci <= ri
    for j in range(q_ref.shape[1] // hd):
        sl = slice(j * hd, (j + 1) * hd)
        q = q_ref[:, sl]
        k = k_ref[:, sl]
        v = v_ref[:, sl]
        s = lax.dot_general(q, k, (((1,), (1,)), ((), ())),
                            preferred_element_type=jnp.float32) * scale
        s = jnp.where(causal, s, jnp.float32(-1e9))
        m = jnp.max(s, axis=-1, keepdims=True)
        e = jnp.exp(s - m)
        p = e / jnp.sum(e, axis=-1, keepdims=True)
        o_ref[:, sl] = lax.dot_general(p, v, (((1,), (0,)), ((), ())),
                                       preferred_element_type=jnp.float32)


def _k3_proj_router(attn_ref, pw_ref, x_ref, g_ref, b_ref, rw_ref,
                    h_ref, ln2_ref, tp_ref, ti_ref):
    h = x_ref[...] + lax.dot_general(attn_ref[...], pw_ref[...],
                                     (((1,), (1,)), ((), ())),
                                     preferred_element_type=jnp.float32)
    h_ref[...] = h
    mu = jnp.mean(h, axis=-1, keepdims=True)
    var = jnp.mean((h - mu) ** 2, axis=-1, keepdims=True)
    ln = (h - mu) / jnp.sqrt(var + 1e-5) * g_ref[...] + b_ref[...]
    ln2_ref[...] = ln
    logits = lax.dot_general(ln, rw_ref[...], (((1,), (1,)), ((), ())),
                             preferred_element_type=jnp.float32)
    mx = jnp.max(logits, axis=-1, keepdims=True)
    ex = jnp.exp(logits - mx)
    pr = ex / jnp.sum(ex, axis=-1, keepdims=True)
    E = pr.shape[-1]
    idx = lax.broadcasted_iota(jnp.int32, pr.shape, 1)
    p1 = jnp.max(pr, axis=-1, keepdims=True)
    i1 = jnp.min(jnp.where(pr == p1, idx, E), axis=-1, keepdims=True)
    pr2 = jnp.where(idx == i1, jnp.float32(-1.0), pr)
    p2 = jnp.max(pr2, axis=-1, keepdims=True)
    i2 = jnp.min(jnp.where(pr2 == p2, idx, E), axis=-1, keepdims=True)
    z = jnp.zeros((pr.shape[0], 6), jnp.float32)
    tp_ref[...] = jnp.concatenate([p1, p2, z], axis=-1)
    ti_ref[...] = jnp.concatenate([i1, i2, z.astype(jnp.int32)], axis=-1)


def _k5_moe(ue_ref, um_ref, uf_ref, uv_ref, off_ref,
            x_ref, w1_ref, w2_ref, o_ref):
    i = pl.program_id(0)

    @pl.when(uf_ref[i] == 1)
    def _():
        o_ref[...] = jnp.zeros_like(o_ref)

    @pl.when(uv_ref[i] == 1)
    def _():
        x = x_ref[...]
        mid = lax.dot_general(x, w1_ref[0], (((1,), (0,)), ((), ())),
                              preferred_element_type=jnp.float32)
        mid = mid * 0.5 * (1.0 + lax.erf(mid * 0.7071067811865476))
        out = lax.dot_general(mid, w2_ref[0], (((1,), (0,)), ((), ())),
                              preferred_element_type=jnp.float32)
        e = ue_ref[i]
        lo = off_ref[e]
        hi = off_ref[e + 1]
        rows = um_ref[i] * MT + lax.broadcasted_iota(jnp.int32, (MT, 1), 0)
        act = (rows >= lo) & (rows < hi)
        o_ref[...] += jnp.where(act, out, 0.0)


def _k7_combine(h_ref, tp_ref, r_ref, o_ref):
    o_ref[...] = (h_ref[...]
                  + tp_ref[:, 0:1] * r_ref[:, 0, :]
                  + tp_ref[:, 1:2] * r_ref[:, 1, :])


def _sc_gather(table, idx):
    V, D = table.shape
    Bn = idx.shape[0]
    info = plsc.get_sparse_core_info()
    NC, NS = info.num_cores, info.num_subcores
    NW = NC * NS
    rpw = Bn // NW
    CH = min(64, rpw)
    nch = rpw // CH
    mesh = plsc.VectorSubcoreMesh(core_axis_name="c", subcore_axis_name="s")

    @functools.partial(
        pl.kernel, mesh=mesh,
        out_type=jax.ShapeDtypeStruct((Bn, D), jnp.float32),
        scratch_types=[
            pltpu.VMEM((CH,), jnp.int32),
            pltpu.VMEM((CH, D), jnp.float32),
            pltpu.SemaphoreType.DMA,
        ],
    )
    def g(table_hbm, idx_hbm, out_hbm, idx_v, rows_v, sem):
        wid = lax.axis_index("s") * NC + lax.axis_index("c")
        for c in range(nch):
            base = wid * rpw + c * CH
            pltpu.sync_copy(idx_hbm.at[pl.ds(base, CH)], idx_v)
            pltpu.async_copy(table_hbm.at[idx_v], rows_v, sem).wait()
            pltpu.sync_copy(rows_v, out_hbm.at[pl.ds(base, CH)])

    return g(table, idx)


def kernel(hidden_states, ln1_weight, ln1_bias, ln2_weight, ln2_bias,
           qkv_weight, proj_weight, router_weight, moe_w1, moe_w2):
    S, B, H = hidden_states.shape
    E = router_weight.shape[0]
    F = moe_w1.shape[2]
    T = S * B
    P = T * TOPK
    hd = H // NH
    x2d = hidden_states.reshape(T, H)
    g1 = ln1_weight.reshape(1, H)
    b1 = ln1_bias.reshape(1, H)
    g2 = ln2_weight.reshape(1, H)
    b2 = ln2_bias.reshape(1, H)

    qkv = pl.pallas_call(
        _k1_ln_qkv,
        grid=(T // TM,),
        in_specs=[
            pl.BlockSpec((TM, H), lambda i: (i, 0)),
            pl.BlockSpec((3 * H, H), lambda i: (0, 0)),
            pl.BlockSpec((1, H), lambda i: (0, 0)),
            pl.BlockSpec((1, H), lambda i: (0, 0)),
        ],
        out_specs=pl.BlockSpec((TM, 3 * H), lambda i: (i, 0)),
        out_shape=jax.ShapeDtypeStruct((T, 3 * H), jnp.float32),
    )(x2d, qkv_weight, g1, b1)

    hpg = max(1, 128 // hd)
    hb = hpg * hd
    ng = NH // hpg
    attn = pl.pallas_call(
        functools.partial(_k2_attn, scale=1.0 / (hd ** 0.5), hd=hd),
        grid=(ng,),
        in_specs=[
            pl.BlockSpec((T, hb), lambda h: (0, h)),
            pl.BlockSpec((T, hb), lambda h: (0, ng + h)),
            pl.BlockSpec((T, hb), lambda h: (0, 2 * ng + h)),
        ],
        out_specs=pl.BlockSpec((T, hb), lambda h: (0, h)),
        out_shape=jax.ShapeDtypeStruct((T, H), jnp.float32),
    )(qkv, qkv, qkv)

    h, ln2, top_p, top_i = pl.pallas_call(
        _k3_proj_router,
        grid=(T // TM,),
        in_specs=[
            pl.BlockSpec((TM, H), lambda i: (i, 0)),
            pl.BlockSpec((H, H), lambda i: (0, 0)),
            pl.BlockSpec((TM, H), lambda i: (i, 0)),
            pl.BlockSpec((1, H), lambda i: (0, 0)),
            pl.BlockSpec((1, H), lambda i: (0, 0)),
            pl.BlockSpec((E, H), lambda i: (0, 0)),
        ],
        out_specs=[
            pl.BlockSpec((TM, H), lambda i: (i, 0)),
            pl.BlockSpec((TM, H), lambda i: (i, 0)),
            pl.BlockSpec((TM, 8), lambda i: (i, 0)),
            pl.BlockSpec((TM, 8), lambda i: (i, 0)),
        ],
        out_shape=[
            jax.ShapeDtypeStruct((T, H), jnp.float32),
            jax.ShapeDtypeStruct((T, H), jnp.float32),
            jax.ShapeDtypeStruct((T, 8), jnp.float32),
            jax.ShapeDtypeStruct((T, 8), jnp.int32),
        ],
    )(attn, proj_weight, x2d, g2, b2, router_weight)

    flat_expert = top_i[:, :TOPK].reshape(-1)
    sorted_indices = jnp.argsort(flat_expert, stable=True)
    token_ids = sorted_indices // TOPK
    restore = jnp.argsort(sorted_indices)
    sizes = jnp.bincount(flat_expert, length=E)
    off = jnp.concatenate([jnp.zeros(1, jnp.int32),
                           jnp.cumsum(sizes).astype(jnp.int32)])
    n_mt = P // MT
    UNITS = n_mt + E - 1
    t0 = off[:E] // MT
    t1 = jnp.maximum(off[1:] - 1, 0) // MT
    nu = jnp.where(sizes > 0, t1 - t0 + 1, 0).astype(jnp.int32)
    ucum = jnp.concatenate([jnp.zeros(1, jnp.int32),
                            jnp.cumsum(nu).astype(jnp.int32)])
    W = ucum[E]
    ii = jnp.arange(UNITS, dtype=jnp.int32)
    eidx = jnp.clip(jnp.searchsorted(ucum, ii, side='right') - 1, 0, E - 1)
    eidx = eidx.astype(jnp.int32)
    valid = ii < W
    e_last = jnp.max(jnp.where(valid, eidx, -1)).astype(jnp.int32)
    um = jnp.where(valid, t0[eidx] + (ii - ucum[eidx]), n_mt - 1)
    um = jnp.clip(um, 0, n_mt - 1).astype(jnp.int32)
    ue = jnp.where(valid, eidx, e_last).astype(jnp.int32)
    uf = jnp.concatenate([jnp.ones(1, jnp.int32),
                          (um[1:] != um[:-1]).astype(jnp.int32)])
    uv = valid.astype(jnp.int32)

    xg = _sc_gather(ln2, token_ids.astype(jnp.int32))

    eout = pl.pallas_call(
        _k5_moe,
        grid_spec=pltpu.PrefetchScalarGridSpec(
            num_scalar_prefetch=5,
            grid=(UNITS,),
            in_specs=[
                pl.BlockSpec((MT, H), lambda i, ue, um, uf, uv, off: (um[i], 0)),
                pl.BlockSpec((1, H, F), lambda i, ue, um, uf, uv, off: (ue[i], 0, 0)),
                pl.BlockSpec((1, F, H), lambda i, ue, um, uf, uv, off: (ue[i], 0, 0)),
            ],
            out_specs=pl.BlockSpec((MT, H), lambda i, ue, um, uf, uv, off: (um[i], 0)),
        ),
        out_shape=jax.ShapeDtypeStruct((P, H), jnp.float32),
    )(ue, um, uf, uv, off, xg, moe_w1, moe_w2)

    r = _sc_gather(eout, restore.astype(jnp.int32)).reshape(T, TOPK, H)
    out = pl.pallas_call(
        _k7_combine,
        grid=(T // TM,),
        in_specs=[
            pl.BlockSpec((TM, H), lambda i: (i, 0)),
            pl.BlockSpec((TM, 8), lambda i: (i, 0)),
            pl.BlockSpec((TM, TOPK, H), lambda i: (i, 0, 0)),
        ],
        out_specs=pl.BlockSpec((TM, H), lambda i: (i, 0)),
        out_shape=jax.ShapeDtypeStruct((T, H), jnp.float32),
    )(h, top_p, r)
    return out.reshape(S, B, H)


# speedup vs baseline: 8.9463x; 1.0013x over previous
import functools
import jax
import jax.numpy as jnp
from jax import lax
from jax.experimental import pallas as pl
from jax.experimental.pallas import tpu as pltpu
from jax.experimental.pallas import tpu_sc as plsc

NH = 16
TOPK = 2
TM = 256
MT = 128


def _k1_ln_qkv(x_ref, w_ref, g_ref, b_ref, o_ref):
    x = x_ref[...]
    mu = jnp.mean(x, axis=-1, keepdims=True)
    var = jnp.mean((x - mu) ** 2, axis=-1, keepdims=True)
    ln = (x - mu) / jnp.sqrt(var + 1e-5) * g_ref[...] + b_ref[...]
    o_ref[...] = lax.dot_general(ln, w_ref[...], (((1,), (1,)), ((), ())),
                                 preferred_element_type=jnp.float32, precision=lax.Precision.DEFAULT)


def _k2_attn(q_ref, k_ref, v_ref, o_ref, *, scale, hd):
    S = q_ref.shape[0]
    ri = lax.broadcasted_iota(jnp.int32, (S, S), 0)
    ci = lax.broadcasted_iota(jnp.int32, (S, S), 1)
    causal = ci <= ri
    for j in range(q_ref.shape[1] // hd):
        sl = slice(j * hd, (j + 1) * hd)
        q = q_ref[:, sl]
        k = k_ref[:, sl]
        v = v_ref[:, sl]
        s = lax.dot_general(q, k, (((1,), (1,)), ((), ())),
                            preferred_element_type=jnp.float32, precision=lax.Precision.DEFAULT) * scale
        s = jnp.where(causal, s, jnp.float32(-1e9))
        m = jnp.max(s, axis=-1, keepdims=True)
        e = jnp.exp(s - m)
        p = e / jnp.sum(e, axis=-1, keepdims=True)
        o_ref[:, sl] = lax.dot_general(p, v, (((1,), (0,)), ((), ())),
                                       preferred_element_type=jnp.float32, precision=lax.Precision.DEFAULT)


def _k3_proj_router(attn_ref, pw_ref, x_ref, g_ref, b_ref, rw_ref,
                    h_ref, ln2_ref, tp_ref, ti_ref):
    h = x_ref[...] + lax.dot_general(attn_ref[...], pw_ref[...],
                                     (((1,), (1,)), ((), ())),
                                     preferred_element_type=jnp.float32, precision=lax.Precision.DEFAULT)
    h_ref[...] = h
    mu = jnp.mean(h, axis=-1, keepdims=True)
    var = jnp.mean((h - mu) ** 2, axis=-1, keepdims=True)
    ln = (h - mu) / jnp.sqrt(var + 1e-5) * g_ref[...] + b_ref[...]
    ln2_ref[...] = ln
    logits = lax.dot_general(ln, rw_ref[...], (((1,), (1,)), ((), ())),
                             preferred_element_type=jnp.float32, precision=lax.Precision.DEFAULT)
    mx = jnp.max(logits, axis=-1, keepdims=True)
    ex = jnp.exp(logits - mx)
    pr = ex / jnp.sum(ex, axis=-1, keepdims=True)
    E = pr.shape[-1]
    idx = lax.broadcasted_iota(jnp.int32, pr.shape, 1)
    p1 = jnp.max(pr, axis=-1, keepdims=True)
    i1 = jnp.min(jnp.where(pr == p1, idx, E), axis=-1, keepdims=True)
    pr2 = jnp.where(idx == i1, jnp.float32(-1.0), pr)
    p2 = jnp.max(pr2, axis=-1, keepdims=True)
    i2 = jnp.min(jnp.where(pr2 == p2, idx, E), axis=-1, keepdims=True)
    z = jnp.zeros((pr.shape[0], 6), jnp.float32)
    tp_ref[...] = jnp.concatenate([p1, p2, z], axis=-1)
    ti_ref[...] = jnp.concatenate([i1, i2, z.astype(jnp.int32)], axis=-1)


def _k5_moe(ue_ref, um_ref, uf_ref, uv_ref, off_ref,
            x_ref, w1_ref, w2_ref, o_ref):
    i = pl.program_id(0)

    @pl.when(uf_ref[i] == 1)
    def _():
        o_ref[...] = jnp.zeros_like(o_ref)

    @pl.when(uv_ref[i] == 1)
    def _():
        x = x_ref[...]
        mid = lax.dot_general(x, w1_ref[0], (((1,), (0,)), ((), ())),
                              preferred_element_type=jnp.float32, precision=lax.Precision.DEFAULT)
        mid = mid * 0.5 * (1.0 + lax.erf(mid * 0.7071067811865476))
        out = lax.dot_general(mid, w2_ref[0], (((1,), (0,)), ((), ())),
                              preferred_element_type=jnp.float32, precision=lax.Precision.DEFAULT)
        e = ue_ref[i]
        lo = off_ref[e]
        hi = off_ref[e + 1]
        rows = um_ref[i] * MT + lax.broadcasted_iota(jnp.int32, (MT, 1), 0)
        act = (rows >= lo) & (rows < hi)
        o_ref[...] += jnp.where(act, out, 0.0)


def _k7_combine(h_ref, tp_ref, r_ref, o_ref):
    o_ref[...] = (h_ref[...]
                  + tp_ref[:, 0:1] * r_ref[:, 0, :]
                  + tp_ref[:, 1:2] * r_ref[:, 1, :])


def _sc_gather(table, idx):
    V, D = table.shape
    Bn = idx.shape[0]
    info = plsc.get_sparse_core_info()
    NC, NS = info.num_cores, info.num_subcores
    NW = NC * NS
    rpw = Bn // NW
    CH = min(64, rpw)
    nch = rpw // CH
    mesh = plsc.VectorSubcoreMesh(core_axis_name="c", subcore_axis_name="s")

    @functools.partial(
        pl.kernel, mesh=mesh,
        out_type=jax.ShapeDtypeStruct((Bn, D), jnp.float32),
        scratch_types=[
            pltpu.VMEM((CH,), jnp.int32),
            pltpu.VMEM((CH, D), jnp.float32),
            pltpu.SemaphoreType.DMA,
        ],
    )
    def g(table_hbm, idx_hbm, out_hbm, idx_v, rows_v, sem):
        wid = lax.axis_index("s") * NC + lax.axis_index("c")
        for c in range(nch):
            base = wid * rpw + c * CH
            pltpu.sync_copy(idx_hbm.at[pl.ds(base, CH)], idx_v)
            pltpu.async_copy(table_hbm.at[idx_v], rows_v, sem).wait()
            pltpu.sync_copy(rows_v, out_hbm.at[pl.ds(base, CH)])

    return g(table, idx)


def kernel(hidden_states, ln1_weight, ln1_bias, ln2_weight, ln2_bias,
           qkv_weight, proj_weight, router_weight, moe_w1, moe_w2):
    S, B, H = hidden_states.shape
    E = router_weight.shape[0]
    F = moe_w1.shape[2]
    T = S * B
    P = T * TOPK
    hd = H // NH
    x2d = hidden_states.reshape(T, H)
    g1 = ln1_weight.reshape(1, H)
    b1 = ln1_bias.reshape(1, H)
    g2 = ln2_weight.reshape(1, H)
    b2 = ln2_bias.reshape(1, H)

    qkv = pl.pallas_call(
        _k1_ln_qkv,
        grid=(T // TM,),
        in_specs=[
            pl.BlockSpec((TM, H), lambda i: (i, 0)),
            pl.BlockSpec((3 * H, H), lambda i: (0, 0)),
            pl.BlockSpec((1, H), lambda i: (0, 0)),
            pl.BlockSpec((1, H), lambda i: (0, 0)),
        ],
        out_specs=pl.BlockSpec((TM, 3 * H), lambda i: (i, 0)),
        out_shape=jax.ShapeDtypeStruct((T, 3 * H), jnp.float32),
    )(x2d, qkv_weight, g1, b1)

    hpg = max(1, 128 // hd)
    hb = hpg * hd
    ng = NH // hpg
    attn = pl.pallas_call(
        functools.partial(_k2_attn, scale=1.0 / (hd ** 0.5), hd=hd),
        grid=(ng,),
        in_specs=[
            pl.BlockSpec((T, hb), lambda h: (0, h)),
            pl.BlockSpec((T, hb), lambda h: (0, ng + h)),
            pl.BlockSpec((T, hb), lambda h: (0, 2 * ng + h)),
        ],
        out_specs=pl.BlockSpec((T, hb), lambda h: (0, h)),
        out_shape=jax.ShapeDtypeStruct((T, H), jnp.float32),
    )(qkv, qkv, qkv)

    h, ln2, top_p, top_i = pl.pallas_call(
        _k3_proj_router,
        grid=(T // TM,),
        in_specs=[
            pl.BlockSpec((TM, H), lambda i: (i, 0)),
            pl.BlockSpec((H, H), lambda i: (0, 0)),
            pl.BlockSpec((TM, H), lambda i: (i, 0)),
            pl.BlockSpec((1, H), lambda i: (0, 0)),
            pl.BlockSpec((1, H), lambda i: (0, 0)),
            pl.BlockSpec((E, H), lambda i: (0, 0)),
        ],
        out_specs=[
            pl.BlockSpec((TM, H), lambda i: (i, 0)),
            pl.BlockSpec((TM, H), lambda i: (i, 0)),
            pl.BlockSpec((TM, 8), lambda i: (i, 0)),
            pl.BlockSpec((TM, 8), lambda i: (i, 0)),
        ],
        out_shape=[
            jax.ShapeDtypeStruct((T, H), jnp.float32),
            jax.ShapeDtypeStruct((T, H), jnp.float32),
            jax.ShapeDtypeStruct((T, 8), jnp.float32),
            jax.ShapeDtypeStruct((T, 8), jnp.int32),
        ],
    )(attn, proj_weight, x2d, g2, b2, router_weight)

    flat_expert = top_i[:, :TOPK].reshape(-1)
    sorted_indices = jnp.argsort(flat_expert, stable=True)
    token_ids = sorted_indices // TOPK
    restore = jnp.argsort(sorted_indices)
    sizes = jnp.bincount(flat_expert, length=E)
    off = jnp.concatenate([jnp.zeros(1, jnp.int32),
                           jnp.cumsum(sizes).astype(jnp.int32)])
    n_mt = P // MT
    UNITS = n_mt + E - 1
    t0 = off[:E] // MT
    t1 = jnp.maximum(off[1:] - 1, 0) // MT
    nu = jnp.where(sizes > 0, t1 - t0 + 1, 0).astype(jnp.int32)
    ucum = jnp.concatenate([jnp.zeros(1, jnp.int32),
                            jnp.cumsum(nu).astype(jnp.int32)])
    W = ucum[E]
    ii = jnp.arange(UNITS, dtype=jnp.int32)
    eidx = jnp.clip(jnp.searchsorted(ucum, ii, side='right') - 1, 0, E - 1)
    eidx = eidx.astype(jnp.int32)
    valid = ii < W
    e_last = jnp.max(jnp.where(valid, eidx, -1)).astype(jnp.int32)
    um = jnp.where(valid, t0[eidx] + (ii - ucum[eidx]), n_mt - 1)
    um = jnp.clip(um, 0, n_mt - 1).astype(jnp.int32)
    ue = jnp.where(valid, eidx, e_last).astype(jnp.int32)
    uf = jnp.concatenate([jnp.ones(1, jnp.int32),
                          (um[1:] != um[:-1]).astype(jnp.int32)])
    uv = valid.astype(jnp.int32)

    xg = _sc_gather(ln2, token_ids.astype(jnp.int32))

    eout = pl.pallas_call(
        _k5_moe,
        grid_spec=pltpu.PrefetchScalarGridSpec(
            num_scalar_prefetch=5,
            grid=(UNITS,),
            in_specs=[
                pl.BlockSpec((MT, H), lambda i, ue, um, uf, uv, off: (um[i], 0)),
                pl.BlockSpec((1, H, F), lambda i, ue, um, uf, uv, off: (ue[i], 0, 0)),
                pl.BlockSpec((1, F, H), lambda i, ue, um, uf, uv, off: (ue[i], 0, 0)),
            ],
            out_specs=pl.BlockSpec((MT, H), lambda i, ue, um, uf, uv, off: (um[i], 0)),
        ),
        out_shape=jax.ShapeDtypeStruct((P, H), jnp.float32),
    )(ue, um, uf, uv, off, xg, moe_w1, moe_w2)

    r = _sc_gather(eout, restore.astype(jnp.int32)).reshape(T, TOPK, H)
    out = pl.pallas_call(
        _k7_combine,
        grid=(T // TM,),
        in_specs=[
            pl.BlockSpec((TM, H), lambda i: (i, 0)),
            pl.BlockSpec((TM, 8), lambda i: (i, 0)),
            pl.BlockSpec((TM, TOPK, H), lambda i: (i, 0, 0)),
        ],
        out_specs=pl.BlockSpec((TM, H), lambda i: (i, 0)),
        out_shape=jax.ShapeDtypeStruct((T, H), jnp.float32),
    )(h, top_p, r)
    return out.reshape(S, B, H)
